# baseline (speedup 1.0000x reference)
import jax
import jax.numpy as jnp
from jax.experimental import pallas as pl
from jax.experimental.pallas import tpu as pltpu

F = 64
S = F * F
R1 = 59 * F + 60
R2 = 58 * F + 58 + 1
R3 = 50 * F + 50 + 1
R4 = 42 * F + 42 + 1
R5 = 40 * F + 40 + 1
W1 = 3840
W2 = 3328
W3 = 2816


def _cnn_kernel(x_ref, w1_ref, b1_ref, w2_ref, b2_ref, w3_ref, b3_ref,
                sel_ref, wl_ref, bl_ref, o_ref,
                c1o, p1o, c2o, c3o, slab1, slab2, slab3, flat):
    slab1[0:32, 0:W1] = jnp.zeros((32, W1), jnp.bfloat16)
    for dy in range(2):
        for dx in range(5):
            base = 64 * dy + dx
            win = x_ref[0, :, base:S]
            for kk in range(3):
                row = dy + 2 * kk
                if row >= 5:
                    continue
                t = row * 5 + dx
                slab1[t:t + 1, 0:R1] = \
                    win[:, 128 * kk:128 * kk + R1].astype(jnp.bfloat16)
    b1 = b1_ref[...]
    y1 = jnp.dot(w1_ref[...], slab1[0:32, 0:R1],
                 preferred_element_type=jnp.float32)
    c1o[0:16, 0:R1] = jnp.maximum(y1 + b1, 0.0)

    p1o[0:16, 0:R2] = jnp.maximum(
        jnp.maximum(c1o[0:16, 0:R2], c1o[0:16, 1:R2 + 1]),
        jnp.maximum(c1o[0:16, F:R2 + F], c1o[0:16, F + 1:R2 + F + 1])
    ).astype(jnp.bfloat16)

    b2 = b2_ref[...]
    for dx in range(5):
        tmp = p1o[0:16, 2 * dx:2 * dx + R3 + 512]
        for dy in range(5):
            t = dy * 5 + dx
            slab2[t * 16:(t + 1) * 16, 0:R3] = tmp[:, 128 * dy:128 * dy + R3]
    y2 = jnp.dot(w2_ref[...], slab2[0:400, 0:R3],
                 preferred_element_type=jnp.float32)
    c2o[0:32, 0:R3] = jnp.maximum(y2 + b2, 0.0).astype(jnp.bfloat16)

    b3 = b3_ref[...]
    for dx in range(5):
        tmp = c2o[0:32, 2 * dx:2 * dx + R4 + 512]
        for dy in range(5):
            t = dy * 5 + dx
            slab3[t * 32:(t + 1) * 32, 0:R4] = tmp[:, 128 * dy:128 * dy + R4]
    y3 = jnp.dot(w3_ref[...], slab3[0:800, 0:R4],
                 preferred_element_type=jnp.float32)
    c3o[0:32, 0:R4] = jnp.maximum(y3 + b3, 0.0).astype(jnp.bfloat16)

    p2v = jnp.maximum(
        jnp.maximum(c3o[0:32, 0:R5], c3o[0:32, 2:R5 + 2]),
        jnp.maximum(c3o[0:32, 2 * F:R5 + 2 * F],
                    c3o[0:32, 2 * F + 2:R5 + 2 * F + 2]))

    dense = jnp.dot(p2v, sel_ref[...],
                    preferred_element_type=jnp.float32)

    for c in range(32):
        flat[0:1, 128 * c:128 * (c + 1)] = dense[c:c + 1, :].astype(jnp.bfloat16)
    out = jnp.dot(flat[...], wl_ref[...],
                  preferred_element_type=jnp.float32) + bl_ref[...]
    o_ref[0, :, :] = out


def kernel(x, w1, b1, w2, b2, w3, b3, wl, bl):
    n = x.shape[0]
    k = bl.shape[0]

    x_flat = x.astype(jnp.float32).reshape(n, 1, S)
    w1r = jnp.pad(jnp.transpose(w1, (0, 2, 3, 1)).reshape(16, 25),
                  ((0, 0), (0, 7))).astype(jnp.bfloat16)
    w2r = jnp.transpose(w2, (0, 2, 3, 1)).reshape(32, 400).astype(jnp.bfloat16)
    w3r = jnp.transpose(w3, (0, 2, 3, 1)).reshape(32, 800).astype(jnp.bfloat16)
    b1r = b1.reshape(16, 1).astype(jnp.float32)
    b2r = b2.reshape(32, 1).astype(jnp.float32)
    b3r = b3.reshape(32, 1).astype(jnp.float32)
    blr = bl.reshape(1, k).astype(jnp.float32)

    wlp = jnp.pad(wl.T.reshape(32, 121, k),
                  ((0, 0), (0, 7), (0, 0))).reshape(32 * 128, k).astype(jnp.bfloat16)

    yy, xx = jnp.meshgrid(jnp.arange(11), jnp.arange(11), indexing="ij")
    lanes = (4 * (yy * F + xx)).reshape(-1)
    cols = (yy * 11 + xx).reshape(-1)
    sel = jnp.zeros((R5, 128), jnp.float32).at[lanes, cols].set(1.0)
    sel = sel.astype(jnp.bfloat16)

    flops = 2 * n * (16 * 25 * R1 + 32 * 400 * R3 + 32 * 800 * R4
                     + 32 * 128 * R5 + 4096 * k)
    bytes_accessed = (4 * n * S + 4 * n * k
                      + 2 * (32 * 400 + 32 * 800 + R5 * 128 + 4096 * k)
                      + 4 * (16 * 25 + 16 + 32 + 32 + k))

    out = pl.pallas_call(
        _cnn_kernel,
        out_shape=jax.ShapeDtypeStruct((n, 1, k), jnp.float32),
        grid=(n,),
        in_specs=[
            pl.BlockSpec((1, 1, S), lambda i: (i, 0, 0)),
            pl.BlockSpec((16, 32), lambda i: (0, 0)),
            pl.BlockSpec((16, 1), lambda i: (0, 0)),
            pl.BlockSpec((32, 400), lambda i: (0, 0)),
            pl.BlockSpec((32, 1), lambda i: (0, 0)),
            pl.BlockSpec((32, 800), lambda i: (0, 0)),
            pl.BlockSpec((32, 1), lambda i: (0, 0)),
            pl.BlockSpec((R5, 128), lambda i: (0, 0)),
            pl.BlockSpec((32 * 128, k), lambda i: (0, 0)),
            pl.BlockSpec((1, k), lambda i: (0, 0)),
        ],
        out_specs=pl.BlockSpec((1, 1, k), lambda i: (i, 0, 0)),
        scratch_shapes=[
            pltpu.VMEM((16, S), jnp.float32),
            pltpu.VMEM((16, S), jnp.bfloat16),
            pltpu.VMEM((32, S), jnp.bfloat16),
            pltpu.VMEM((32, S), jnp.bfloat16),
            pltpu.VMEM((32, W1), jnp.bfloat16),
            pltpu.VMEM((400, W2), jnp.bfloat16),
            pltpu.VMEM((800, W3), jnp.bfloat16),
            pltpu.VMEM((1, 32 * 128), jnp.bfloat16),
        ],
        compiler_params=pltpu.CompilerParams(
            dimension_semantics=("parallel",),
            vmem_limit_bytes=32 * 1024 * 1024),
        cost_estimate=pl.CostEstimate(flops=flops, transcendentals=0,
                                      bytes_accessed=bytes_accessed),
    )(x_flat, w1r, b1r, w2r, b2r, w3r, b3r, sel, wlp, blr)
    return out.reshape(n, k)


# speedup vs baseline: 1.0018x; 1.0018x over previous
import jax
import jax.numpy as jnp
from jax.experimental import pallas as pl
from jax.experimental.pallas import tpu as pltpu

F = 64
S = F * F
R1 = 59 * F + 60
R2 = 58 * F + 58 + 1
R3 = 50 * F + 50 + 1
R4 = 42 * F + 42 + 1
R5 = 40 * F + 40 + 1
W1 = 3840
W2 = 3328
W3 = 2816


def _cnn_kernel(x_ref, w1_ref, b1_ref, w2_ref, b2_ref, w3_ref, b3_ref,
                sel_ref, wl_ref, bl_ref, o_ref,
                c1o, p1o, c2o, c3o, p2, slab1, slab2, slab3, flat):
    slab1[0:32, 0:W1] = jnp.zeros((32, W1), jnp.bfloat16)
    for dy in range(2):
        for dx in range(5):
            base = 64 * dy + dx
            win = x_ref[0, :, base:S]
            for kk in range(3):
                row = dy + 2 * kk
                if row >= 5:
                    continue
                t = row * 5 + dx
                slab1[t:t + 1, 0:R1] = \
                    win[:, 128 * kk:128 * kk + R1].astype(jnp.bfloat16)
    b1 = b1_ref[...]
    y1 = jnp.dot(w1_ref[...], slab1[0:32, 0:R1],
                 preferred_element_type=jnp.float32)
    c1o[0:16, 0:R1] = jnp.maximum(y1 + b1, 0.0)

    p1o[0:16, 0:R2] = jnp.maximum(
        jnp.maximum(c1o[0:16, 0:R2], c1o[0:16, 1:R2 + 1]),
        jnp.maximum(c1o[0:16, F:R2 + F], c1o[0:16, F + 1:R2 + F + 1])
    ).astype(jnp.bfloat16)

    b2 = b2_ref[...]
    for dx in range(5):
        tmp = p1o[0:16, 2 * dx:2 * dx + R3 + 512]
        for dy in range(5):
            t = dy * 5 + dx
            slab2[t * 16:(t + 1) * 16, 0:R3] = tmp[:, 128 * dy:128 * dy + R3]
    y2 = jnp.dot(w2_ref[...], slab2[0:400, 0:R3],
                 preferred_element_type=jnp.float32)
    c2o[0:32, 0:R3] = jnp.maximum(y2 + b2, 0.0).astype(jnp.bfloat16)

    b3 = b3_ref[...]
    for dx in range(5):
        tmp = c2o[0:32, 2 * dx:2 * dx + R4 + 512]
        for dy in range(5):
            t = dy * 5 + dx
            slab3[t * 32:(t + 1) * 32, 0:R4] = tmp[:, 128 * dy:128 * dy + R4]
    y3 = jnp.dot(w3_ref[...], slab3[0:800, 0:R4],
                 preferred_element_type=jnp.float32)
    c3o[0:32, 0:R4] = jnp.maximum(y3 + b3, 0.0).astype(jnp.bfloat16)

    p2[0:32, 0:R5] = jnp.maximum(
        jnp.maximum(c3o[0:32, 0:R5], c3o[0:32, 2:R5 + 2]),
        jnp.maximum(c3o[0:32, 2 * F:R5 + 2 * F],
                    c3o[0:32, 2 * F + 2:R5 + 2 * F + 2]))

    dense = jnp.dot(p2[0:32, 0:R5], sel_ref[...],
                    preferred_element_type=jnp.float32)

    for c in range(32):
        flat[0:1, 128 * c:128 * (c + 1)] = dense[c:c + 1, :].astype(jnp.bfloat16)
    out = jnp.dot(flat[...], wl_ref[...],
                  preferred_element_type=jnp.float32) + bl_ref[...]
    o_ref[0, :, :] = out


def kernel(x, w1, b1, w2, b2, w3, b3, wl, bl):
    n = x.shape[0]
    k = bl.shape[0]

    x_flat = x.astype(jnp.float32).reshape(n, 1, S)
    w1r = jnp.pad(jnp.transpose(w1, (0, 2, 3, 1)).reshape(16, 25),
                  ((0, 0), (0, 7))).astype(jnp.bfloat16)
    w2r = jnp.transpose(w2, (0, 2, 3, 1)).reshape(32, 400).astype(jnp.bfloat16)
    w3r = jnp.transpose(w3, (0, 2, 3, 1)).reshape(32, 800).astype(jnp.bfloat16)
    b1r = b1.reshape(16, 1).astype(jnp.float32)
    b2r = b2.reshape(32, 1).astype(jnp.float32)
    b3r = b3.reshape(32, 1).astype(jnp.float32)
    blr = bl.reshape(1, k).astype(jnp.float32)

    wlp = jnp.pad(wl.T.reshape(32, 121, k),
                  ((0, 0), (0, 7), (0, 0))).reshape(32 * 128, k).astype(jnp.bfloat16)

    yy, xx = jnp.meshgrid(jnp.arange(11), jnp.arange(11), indexing="ij")
    lanes = (4 * (yy * F + xx)).reshape(-1)
    cols = (yy * 11 + xx).reshape(-1)
    sel = jnp.zeros((R5, 128), jnp.float32).at[lanes, cols].set(1.0)
    sel = sel.astype(jnp.bfloat16)

    flops = 2 * n * (16 * 25 * R1 + 32 * 400 * R3 + 32 * 800 * R4
                     + 32 * 128 * R5 + 4096 * k)
    bytes_accessed = (4 * n * S + 4 * n * k
                      + 2 * (32 * 400 + 32 * 800 + R5 * 128 + 4096 * k)
                      + 4 * (16 * 25 + 16 + 32 + 32 + k))

    out = pl.pallas_call(
        _cnn_kernel,
        out_shape=jax.ShapeDtypeStruct((n, 1, k), jnp.float32),
        grid=(n,),
        in_specs=[
            pl.BlockSpec((1, 1, S), lambda i: (i, 0, 0)),
            pl.BlockSpec((16, 32), lambda i: (0, 0)),
            pl.BlockSpec((16, 1), lambda i: (0, 0)),
            pl.BlockSpec((32, 400), lambda i: (0, 0)),
            pl.BlockSpec((32, 1), lambda i: (0, 0)),
            pl.BlockSpec((32, 800), lambda i: (0, 0)),
            pl.BlockSpec((32, 1), lambda i: (0, 0)),
            pl.BlockSpec((R5, 128), lambda i: (0, 0)),
            pl.BlockSpec((32 * 128, k), lambda i: (0, 0)),
            pl.BlockSpec((1, k), lambda i: (0, 0)),
        ],
        out_specs=pl.BlockSpec((1, 1, k), lambda i: (i, 0, 0)),
        scratch_shapes=[
            pltpu.VMEM((16, S), jnp.float32),
            pltpu.VMEM((16, S), jnp.bfloat16),
            pltpu.VMEM((32, S), jnp.bfloat16),
            pltpu.VMEM((32, S), jnp.bfloat16),
            pltpu.VMEM((32, 2688), jnp.bfloat16),
            pltpu.VMEM((32, W1), jnp.bfloat16),
            pltpu.VMEM((400, W2), jnp.bfloat16),
            pltpu.VMEM((800, W3), jnp.bfloat16),
            pltpu.VMEM((1, 32 * 128), jnp.bfloat16),
        ],
        compiler_params=pltpu.CompilerParams(
            dimension_semantics=("parallel",),
            vmem_limit_bytes=32 * 1024 * 1024),
        cost_estimate=pl.CostEstimate(flops=flops, transcendentals=0,
                                      bytes_accessed=bytes_accessed),
    )(x_flat, w1r, b1r, w2r, b2r, w3r, b3r, sel, wlp, blr)
    return out.reshape(n, k)
